# baseline (speedup 1.0000x reference)
import jax
import jax.numpy as jnp
from jax.experimental import pallas as pl
from jax.experimental.pallas import tpu as pltpu

_SHAPE = (2, 16384, 1024)
_BLOCK_ROWS = 2048


def _copy_update_kernel(key_ref, in_ref, out_ref):
    out_ref[...] = in_ref[...]

    @pl.when((pl.program_id(0) == 1) & (pl.program_id(1) == 0))
    def _():
        row = jax.lax.broadcasted_iota(jnp.int32, (8, 128), 0)
        col = jax.lax.broadcasted_iota(jnp.int32, (8, 128), 1)
        mask = (row == 0) & (col == 1)
        out_ref[0, 0:8, 0:128] = in_ref[0, 0:8, 0:128] + jnp.where(
            mask, 2.0 * key_ref[0], 0.0
        )


def kernel(key, cache_next):
    grid = (2, _SHAPE[1] // _BLOCK_ROWS)
    block = (1, _BLOCK_ROWS, _SHAPE[2])
    out = pl.pallas_call(
        _copy_update_kernel,
        grid=grid,
        out_shape=jax.ShapeDtypeStruct(_SHAPE, jnp.float32),
        in_specs=[
            pl.BlockSpec(memory_space=pltpu.SMEM),
            pl.BlockSpec(block, lambda i, j: (i, j, 0)),
        ],
        out_specs=pl.BlockSpec(block, lambda i, j: (i, j, 0)),
    )(key, cache_next)
    return key, out
